# baseline (speedup 1.0000x reference)
import functools

import jax
import jax.numpy as jnp
from jax import lax
from jax.experimental import pallas as pl
from jax.experimental.pallas import tpu as pltpu
from jax.experimental.pallas import tpu_sc as plsc

D_MODEL = 64
IDX_MINOR = 128


def _make_gather(B: int):
    info = plsc.get_sparse_core_info()
    nw = info.num_cores * info.num_subcores
    K = 8
    C = K * IDX_MINOR
    b_per_w = B // nw
    chunks_per_w = b_per_w // C
    assert b_per_w % C == 0 and B % nw == 0

    mesh = plsc.VectorSubcoreMesh(core_axis_name="c", subcore_axis_name="s")

    @functools.partial(
        pl.kernel,
        mesh=mesh,
        out_type=jax.ShapeDtypeStruct((B, D_MODEL), jnp.float32),
        scratch_types=[
            pltpu.VMEM((K, IDX_MINOR), jnp.int32),
            pltpu.VMEM((C, D_MODEL), jnp.float32),
            pltpu.SemaphoreType.DMA,
        ],
        compiler_params=pltpu.CompilerParams(use_tc_tiling_on_sc=False),
    )
    def gather_kernel(idx_hbm, table_hbm, out_hbm, idx_v, rows_v, sem):
        wid = lax.axis_index("s") * info.num_cores + lax.axis_index("c")
        idx_row0 = wid * (b_per_w // IDX_MINOR)
        out_row0 = wid * b_per_w

        def body(i, carry):
            pltpu.sync_copy(idx_hbm.at[pl.ds(idx_row0 + i * K, K)], idx_v)
            copies = []
            for j in range(K):
                copies.append(
                    pltpu.async_copy(
                        table_hbm.at[idx_v.at[j]],
                        rows_v.at[pl.ds(j * IDX_MINOR, IDX_MINOR)],
                        sem,
                    )
                )
            for c in copies:
                c.wait()
            pltpu.sync_copy(rows_v, out_hbm.at[pl.ds(out_row0 + i * C, C)])
            return carry

        lax.fori_loop(0, chunks_per_w, body, 0)

    return gather_kernel


def kernel(x, emb_weight):
    orig_shape = x.shape
    B = x.size
    idx2d = x.reshape(B // IDX_MINOR, IDX_MINOR).astype(jnp.int32)
    out = _make_gather(B)(idx2d, emb_weight)
    return out.reshape(*orig_shape, D_MODEL)


# speedup vs baseline: 1.0191x; 1.0191x over previous
import functools

import jax
import jax.numpy as jnp
from jax import lax
from jax.experimental import pallas as pl
from jax.experimental.pallas import tpu as pltpu
from jax.experimental.pallas import tpu_sc as plsc

D_MODEL = 64
IDX_MINOR = 128


def _make_gather(B: int):
    info = plsc.get_sparse_core_info()
    nw = info.num_cores * info.num_subcores
    K = 5
    NB = 2
    C = K * IDX_MINOR
    b_per_w = B // nw
    chunks_per_w = b_per_w // C
    G = chunks_per_w // NB
    assert b_per_w % C == 0 and B % nw == 0 and chunks_per_w % NB == 0

    mesh = plsc.VectorSubcoreMesh(core_axis_name="c", subcore_axis_name="s")

    @functools.partial(
        pl.kernel,
        mesh=mesh,
        out_type=jax.ShapeDtypeStruct((B, D_MODEL), jnp.float32),
        scratch_types=[
            [pltpu.VMEM((K, IDX_MINOR), jnp.int32) for _ in range(NB)],
            [pltpu.VMEM((C, D_MODEL), jnp.float32) for _ in range(NB)],
            [pltpu.SemaphoreType.DMA for _ in range(NB)],
            [pltpu.SemaphoreType.DMA for _ in range(NB)],
        ],
        compiler_params=pltpu.CompilerParams(use_tc_tiling_on_sc=False),
    )
    def gather_kernel(idx_hbm, table_hbm, out_hbm, idx_v, rows_v, gsem, wsem):
        wid = lax.axis_index("s") * info.num_cores + lax.axis_index("c")
        idx_row0 = wid * (b_per_w // IDX_MINOR)
        out_row0 = wid * b_per_w

        def fire(i, b):
            pltpu.sync_copy(idx_hbm.at[pl.ds(idx_row0 + i * K, K)], idx_v[b])
            for j in range(K):
                pltpu.async_copy(
                    table_hbm.at[idx_v[b].at[j]],
                    rows_v[b].at[pl.ds(j * IDX_MINOR, IDX_MINOR)],
                    gsem[b],
                )

        def drain_gather(b):
            for j in range(K):
                pltpu.make_async_copy(
                    table_hbm.at[idx_v[b].at[j]],
                    rows_v[b].at[pl.ds(j * IDX_MINOR, IDX_MINOR)],
                    gsem[b],
                ).wait()

        def write_out(i, b):
            return pltpu.async_copy(
                rows_v[b], out_hbm.at[pl.ds(out_row0 + i * C, C)], wsem[b]
            )

        for b in range(NB):
            fire(b, b)

        def body(g, carry):
            for b in range(NB):
                i = g * NB + b
                drain_gather(b)
                w = write_out(i, b)
                w.wait()
                fire(i + NB, b)
            return carry

        lax.fori_loop(0, G - 1, body, 0)

        for b in range(NB):
            i = (G - 1) * NB + b
            drain_gather(b)
            write_out(i, b).wait()

    return gather_kernel


def kernel(x, emb_weight):
    orig_shape = x.shape
    B = x.size
    idx2d = x.reshape(B // IDX_MINOR, IDX_MINOR).astype(jnp.int32)
    out = _make_gather(B)(idx2d, emb_weight)
    return out.reshape(*orig_shape, D_MODEL)
